# baseline (speedup 1.0000x reference)
import functools

import jax
import jax.numpy as jnp
from jax import lax
from jax.experimental import pallas as pl
from jax.experimental.pallas import tpu as pltpu
from jax.experimental.pallas import tpu_sc as plsc

N = 10000
E = 320000
NC, NS, L = 2, 16, 16
NW = NC * NS
K = 80
NB = 2
C = 128
EPW = C * K
PAD = EPW - E // NW
NP = N + NW * PAD
RPT = 624
TAIL = N - NS * RPT
ZR = 208

_MESH = dict(core_axis_name="c", subcore_axis_name="s",
             num_cores=NC, num_subcores=NS)


def _fill_const(ref, rows, width, value):
    per_row = width // L

    def body(r, _):
        i = r // per_row
        k = (r % per_row) * L
        ref[i, pl.ds(k, L)] = jnp.full((L,), value, jnp.float32)
        return 0

    lax.fori_loop(0, rows * per_row, body, 0)


def _make_agg(F):
    def body(g_hbm, src_hbm, dst_hbm, out_hbm, *refs):
        c = lax.axis_index("c")
        s = lax.axis_index("s")
        wid = s * NC + c
        src_v, dst_v = refs[0], refs[1]
        bufs = refs[2:2 + NB]
        zb = refs[2 + NB]
        out_sh = refs[3 + NB]
        gsem = refs[4 + NB:4 + 2 * NB]
        ssem = refs[4 + 2 * NB:4 + 3 * NB]

        pltpu.sync_copy(src_hbm.at[wid], src_v)
        pltpu.sync_copy(dst_hbm.at[wid], dst_v)

        _fill_const(zb, ZR, F, 0.0)
        base = s * RPT
        for q in range(RPT // ZR):
            pltpu.sync_copy(zb, out_sh.at[pl.ds(base + q * ZR, ZR)])

        @pl.when(s == NS - 1)
        def _():
            pltpu.sync_copy(zb.at[pl.ds(0, TAIL)],
                            out_sh.at[pl.ds(NS * RPT, TAIL)])

        plsc.subcore_barrier()

        def chunk(j, _):
            pltpu.sync_copy(g_hbm.at[src_v.at[j]], bufs[0])
            pltpu.sync_copy(bufs[0], out_sh.at[dst_v.at[j]], add=True)
            return 0

        lax.fori_loop(0, C, chunk, 0)
        plsc.subcore_barrier()

        for q in range(RPT // ZR):
            o = base + q * ZR
            pltpu.sync_copy(out_sh.at[pl.ds(o, ZR)], out_hbm.at[c, pl.ds(o, ZR)])

        @pl.when(s == NS - 1)
        def _():
            pltpu.sync_copy(out_sh.at[pl.ds(NS * RPT, TAIL)],
                            out_hbm.at[c, pl.ds(NS * RPT, TAIL)])

    return pl.kernel(
        body,
        out_type=jax.ShapeDtypeStruct((NC, N, F), jnp.float32),
        mesh=plsc.VectorSubcoreMesh(**_MESH),
        compiler_params=pltpu.CompilerParams(use_tc_tiling_on_sc=False),
        scratch_types=(
            [pltpu.VMEM((C, K), jnp.int32),
             pltpu.VMEM((C, K), jnp.int32)]
            + [pltpu.VMEM((K, F), jnp.float32) for _ in range(NB)]
            + [pltpu.VMEM((ZR, F), jnp.float32),
               pltpu.VMEM_SHARED((NP, F), jnp.float32)]
            + [pltpu.SemaphoreType.DMA for _ in range(2 * NB)]
        ),
    )


def _make_deg():
    F = L

    def body(dst_hbm, out_hbm, dst_v, ones_v, zb, out_sh, sem):
        c = lax.axis_index("c")
        s = lax.axis_index("s")
        wid = s * NC + c

        pltpu.sync_copy(dst_hbm.at[wid], dst_v)
        _fill_const(ones_v, K, F, 1.0)
        _fill_const(zb, ZR, F, 0.0)
        base = s * RPT
        for q in range(RPT // ZR):
            pltpu.sync_copy(zb, out_sh.at[pl.ds(base + q * ZR, ZR)])

        @pl.when(s == NS - 1)
        def _():
            pltpu.sync_copy(zb.at[pl.ds(0, TAIL)],
                            out_sh.at[pl.ds(NS * RPT, TAIL)])

        plsc.subcore_barrier()

        def chunk(j, _):
            pltpu.sync_copy(ones_v, out_sh.at[dst_v.at[j]], add=True)
            return 0

        lax.fori_loop(0, C, chunk, 0)
        plsc.subcore_barrier()

        for q in range(RPT // ZR):
            o = base + q * ZR
            pltpu.sync_copy(out_sh.at[pl.ds(o, ZR)], out_hbm.at[c, pl.ds(o, ZR)])

        @pl.when(s == NS - 1)
        def _():
            pltpu.sync_copy(out_sh.at[pl.ds(NS * RPT, TAIL)],
                            out_hbm.at[c, pl.ds(NS * RPT, TAIL)])

    return pl.kernel(
        body,
        out_type=jax.ShapeDtypeStruct((NC, N, F), jnp.float32),
        mesh=plsc.VectorSubcoreMesh(**_MESH),
        compiler_params=pltpu.CompilerParams(use_tc_tiling_on_sc=False),
        scratch_types=[
            pltpu.VMEM((C, K), jnp.int32),
            pltpu.VMEM((K, F), jnp.float32),
            pltpu.VMEM((ZR, F), jnp.float32),
            pltpu.VMEM_SHARED((NP, F), jnp.float32),
            pltpu.SemaphoreType.DMA,
        ],
    )


_agg32 = _make_agg(32)
_agg64 = _make_agg(64)
_deg = _make_deg()


def _tc(body, out_dim, *args):
    return pl.pallas_call(
        body,
        out_shape=jax.ShapeDtypeStruct((N, out_dim), jnp.float32),
    )(*args)


def _dinv_body(deg_ref, o_ref):
    d = deg_ref[0, :, 0:1] + deg_ref[1, :, 0:1] + 1.0
    o_ref[...] = lax.rsqrt(d)


def _t1_body(x_ref, w_ref, dinv_ref, o_ref):
    o_ref[...] = jnp.dot(x_ref[...], w_ref[...],
                         preferred_element_type=jnp.float32) * dinv_ref[...]


def _t2_body(p_ref, g_ref, b_ref, dinv_ref, o_ref):
    dinv = dinv_ref[...]
    h = jnp.maximum(dinv * (p_ref[0] + p_ref[1] + g_ref[...]) + b_ref[...], 0.0)
    o_ref[...] = h * dinv


def _t3_body(p_ref, g_ref, w_ref, b_ref, dinv_ref, o_ref):
    dinv = dinv_ref[...]
    u = dinv * (p_ref[0] + p_ref[1] + g_ref[...])
    h = jnp.maximum(jnp.dot(u, w_ref[...],
                    preferred_element_type=jnp.float32) + b_ref[...], 0.0)
    o_ref[...] = h * dinv


def _t4_body(p_ref, g_ref, w3_ref, b_ref, w4_ref, dinv_ref, o_ref):
    dinv = dinv_ref[...]
    u = dinv * (p_ref[0] + p_ref[1] + g_ref[...])
    h = jnp.maximum(jnp.dot(u, w3_ref[...],
                    preferred_element_type=jnp.float32) + b_ref[...], 0.0)
    o_ref[...] = jnp.dot(h, w4_ref[...],
                         preferred_element_type=jnp.float32) * dinv


def _t5_body(p_ref, g_ref, b_ref, w_ref, dinv_ref, o_ref):
    dinv = dinv_ref[...]
    h = jnp.maximum(dinv * (p_ref[0] + p_ref[1] + g_ref[...]) + b_ref[...], 0.0)
    o_ref[...] = jnp.dot(h, w_ref[...],
                         preferred_element_type=jnp.float32) * dinv


def _t7_body(p_ref, g_ref, w_ref, b_ref, dinv_ref, o_ref):
    u = dinv_ref[...] * (p_ref[0] + p_ref[1] + g_ref[...])
    z = jnp.dot(u, w_ref[...], preferred_element_type=jnp.float32) + b_ref[...]
    o_ref[...] = jax.nn.sigmoid(z)


def kernel(x, edge_index, W1, b1, W2, b2, W3, b3, W4, b4, W5, b5, W6, b6):
    srcw = edge_index[0].reshape(NW, E // NW)
    dstw = edge_index[1].reshape(NW, E // NW)
    if PAD:
        wids = jnp.arange(NW, dtype=jnp.int32)[:, None]
        js = jnp.arange(PAD, dtype=jnp.int32)[None, :]
        spad = (wids * PAD + js) % N
        dpad = N + wids * PAD + js
        srcw = jnp.concatenate([srcw, spad], axis=1)
        dstw = jnp.concatenate([dstw, dpad], axis=1)
    src3 = srcw.reshape(NW, C, K)
    dst3 = dstw.reshape(NW, C, K)
    b1r, b2r, b3r = b1.reshape(1, -1), b2.reshape(1, -1), b3.reshape(1, -1)
    b4r, b5r, b6r = b4.reshape(1, -1), b5.reshape(1, -1), b6.reshape(1, -1)

    degP = _deg(dst3)
    dinv = _tc(_dinv_body, 1, degP)

    g1 = _tc(_t1_body, 32, x, W1, dinv)
    P = _agg32(g1, src3, dst3)
    g2 = _tc(_t2_body, 32, P, g1, b1r, dinv)
    P = _agg32(g2, src3, dst3)
    g3 = _tc(_t3_body, 64, P, g2, W2, b2r, dinv)
    P = _agg64(g3, src3, dst3)
    g4 = _tc(_t4_body, 64, P, g3, W3, b3r, W4, dinv)
    P = _agg64(g4, src3, dst3)
    g5 = _tc(_t5_body, 32, P, g4, b4r, W5, dinv)
    P = _agg32(g5, src3, dst3)
    g6 = _tc(_t2_body, 32, P, g5, b5r, dinv)
    P = _agg32(g6, src3, dst3)
    return _tc(_t7_body, 128, P, g6, W6, b6r, dinv)


# speedup vs baseline: 1.8883x; 1.8883x over previous
import functools

import jax
import jax.numpy as jnp
from jax import lax
from jax.experimental import pallas as pl
from jax.experimental.pallas import tpu as pltpu
from jax.experimental.pallas import tpu_sc as plsc

N = 10000
E = 320000
NC, NS, L = 2, 16, 16
NW = NC * NS
K = 80
NB = 4
C = 128
EPW = C * K
PAD = EPW - E // NW
NP = N + NW * PAD
RPT = 624
TAIL = N - NS * RPT
ZR = 208

_MESH = dict(core_axis_name="c", subcore_axis_name="s",
             num_cores=NC, num_subcores=NS)


def _fill_const(ref, rows, width, value):
    per_row = width // L

    def body(r, _):
        i = r // per_row
        k = (r % per_row) * L
        ref[i, pl.ds(k, L)] = jnp.full((L,), value, jnp.float32)
        return 0

    lax.fori_loop(0, rows * per_row, body, 0)


def _make_agg(F):
    def body(g_hbm, src_hbm, dst_hbm, out_hbm, *refs):
        c = lax.axis_index("c")
        s = lax.axis_index("s")
        wid = s * NC + c
        src_v, dst_v = refs[0], refs[1]
        bufs = refs[2:2 + NB]
        zb = refs[2 + NB]
        out_sh = refs[3 + NB]
        gsem = refs[4 + NB:4 + 2 * NB]
        ssem = refs[4 + 2 * NB:4 + 3 * NB]

        pltpu.sync_copy(src_hbm.at[wid], src_v)
        pltpu.sync_copy(dst_hbm.at[wid], dst_v)

        _fill_const(zb, ZR, F, 0.0)
        base = s * RPT
        for q in range(RPT // ZR):
            pltpu.sync_copy(zb, out_sh.at[pl.ds(base + q * ZR, ZR)])

        @pl.when(s == NS - 1)
        def _():
            pltpu.sync_copy(zb.at[pl.ds(0, TAIL)],
                            out_sh.at[pl.ds(NS * RPT, TAIL)])

        plsc.subcore_barrier()

        def start_gather(j, i):
            pltpu.async_copy(g_hbm.at[src_v.at[j]], bufs[i], gsem[i])

        def wait_gather(i):
            pltpu.make_async_copy(g_hbm.at[src_v.at[0]], bufs[i],
                                  gsem[i]).wait()

        def start_scatter(j, i):
            pltpu.async_copy(bufs[i], out_sh.at[dst_v.at[j]], ssem[i],
                             add=True)

        def wait_scatter(i):
            pltpu.make_async_copy(bufs[i], out_sh.at[dst_v.at[0]],
                                  ssem[i]).wait()

        for i in range(NB):
            start_gather(i, i)

        def block(t, _):
            for i in range(NB):
                wait_gather(i)
                start_scatter(t * NB + i, i)
            for i in range(NB):
                wait_scatter(i)
                start_gather((t + 1) * NB + i, i)
            return 0

        lax.fori_loop(0, C // NB - 1, block, 0)
        for i in range(NB):
            wait_gather(i)
            start_scatter(C - NB + i, i)
        for i in range(NB):
            wait_scatter(i)
        plsc.subcore_barrier()

        for q in range(RPT // ZR):
            o = base + q * ZR
            pltpu.sync_copy(out_sh.at[pl.ds(o, ZR)], out_hbm.at[c, pl.ds(o, ZR)])

        @pl.when(s == NS - 1)
        def _():
            pltpu.sync_copy(out_sh.at[pl.ds(NS * RPT, TAIL)],
                            out_hbm.at[c, pl.ds(NS * RPT, TAIL)])

    return pl.kernel(
        body,
        out_type=jax.ShapeDtypeStruct((NC, N, F), jnp.float32),
        mesh=plsc.VectorSubcoreMesh(**_MESH),
        compiler_params=pltpu.CompilerParams(use_tc_tiling_on_sc=False),
        scratch_types=(
            [pltpu.VMEM((C, K), jnp.int32),
             pltpu.VMEM((C, K), jnp.int32)]
            + [pltpu.VMEM((K, F), jnp.float32) for _ in range(NB)]
            + [pltpu.VMEM((ZR, F), jnp.float32),
               pltpu.VMEM_SHARED((NP, F), jnp.float32)]
            + [pltpu.SemaphoreType.DMA for _ in range(2 * NB)]
        ),
    )


def _make_deg():
    F = L

    def body(dst_hbm, out_hbm, dst_v, ones_v, zb, out_sh, sem):
        c = lax.axis_index("c")
        s = lax.axis_index("s")
        wid = s * NC + c

        pltpu.sync_copy(dst_hbm.at[wid], dst_v)
        _fill_const(ones_v, K, F, 1.0)
        _fill_const(zb, ZR, F, 0.0)
        base = s * RPT
        for q in range(RPT // ZR):
            pltpu.sync_copy(zb, out_sh.at[pl.ds(base + q * ZR, ZR)])

        @pl.when(s == NS - 1)
        def _():
            pltpu.sync_copy(zb.at[pl.ds(0, TAIL)],
                            out_sh.at[pl.ds(NS * RPT, TAIL)])

        plsc.subcore_barrier()

        def chunk(j, _):
            pltpu.sync_copy(ones_v, out_sh.at[dst_v.at[j]], add=True)
            return 0

        lax.fori_loop(0, C, chunk, 0)
        plsc.subcore_barrier()

        for q in range(RPT // ZR):
            o = base + q * ZR
            pltpu.sync_copy(out_sh.at[pl.ds(o, ZR)], out_hbm.at[c, pl.ds(o, ZR)])

        @pl.when(s == NS - 1)
        def _():
            pltpu.sync_copy(out_sh.at[pl.ds(NS * RPT, TAIL)],
                            out_hbm.at[c, pl.ds(NS * RPT, TAIL)])

    return pl.kernel(
        body,
        out_type=jax.ShapeDtypeStruct((NC, N, F), jnp.float32),
        mesh=plsc.VectorSubcoreMesh(**_MESH),
        compiler_params=pltpu.CompilerParams(use_tc_tiling_on_sc=False),
        scratch_types=[
            pltpu.VMEM((C, K), jnp.int32),
            pltpu.VMEM((K, F), jnp.float32),
            pltpu.VMEM((ZR, F), jnp.float32),
            pltpu.VMEM_SHARED((NP, F), jnp.float32),
            pltpu.SemaphoreType.DMA,
        ],
    )


_agg32 = _make_agg(32)
_agg64 = _make_agg(64)
_deg = _make_deg()


def _tc(body, out_dim, *args):
    return pl.pallas_call(
        body,
        out_shape=jax.ShapeDtypeStruct((N, out_dim), jnp.float32),
    )(*args)


def _dinv_body(deg_ref, o_ref):
    d = deg_ref[0, :, 0:1] + deg_ref[1, :, 0:1] + 1.0
    o_ref[...] = lax.rsqrt(d)


def _t1_body(x_ref, w_ref, dinv_ref, o_ref):
    o_ref[...] = jnp.dot(x_ref[...], w_ref[...],
                         preferred_element_type=jnp.float32) * dinv_ref[...]


def _t2_body(p_ref, g_ref, b_ref, dinv_ref, o_ref):
    dinv = dinv_ref[...]
    h = jnp.maximum(dinv * (p_ref[0] + p_ref[1] + g_ref[...]) + b_ref[...], 0.0)
    o_ref[...] = h * dinv


def _t3_body(p_ref, g_ref, w_ref, b_ref, dinv_ref, o_ref):
    dinv = dinv_ref[...]
    u = dinv * (p_ref[0] + p_ref[1] + g_ref[...])
    h = jnp.maximum(jnp.dot(u, w_ref[...],
                    preferred_element_type=jnp.float32) + b_ref[...], 0.0)
    o_ref[...] = h * dinv


def _t4_body(p_ref, g_ref, w3_ref, b_ref, w4_ref, dinv_ref, o_ref):
    dinv = dinv_ref[...]
    u = dinv * (p_ref[0] + p_ref[1] + g_ref[...])
    h = jnp.maximum(jnp.dot(u, w3_ref[...],
                    preferred_element_type=jnp.float32) + b_ref[...], 0.0)
    o_ref[...] = jnp.dot(h, w4_ref[...],
                         preferred_element_type=jnp.float32) * dinv


def _t5_body(p_ref, g_ref, b_ref, w_ref, dinv_ref, o_ref):
    dinv = dinv_ref[...]
    h = jnp.maximum(dinv * (p_ref[0] + p_ref[1] + g_ref[...]) + b_ref[...], 0.0)
    o_ref[...] = jnp.dot(h, w_ref[...],
                         preferred_element_type=jnp.float32) * dinv


def _t7_body(p_ref, g_ref, w_ref, b_ref, dinv_ref, o_ref):
    u = dinv_ref[...] * (p_ref[0] + p_ref[1] + g_ref[...])
    z = jnp.dot(u, w_ref[...], preferred_element_type=jnp.float32) + b_ref[...]
    o_ref[...] = jax.nn.sigmoid(z)


def kernel(x, edge_index, W1, b1, W2, b2, W3, b3, W4, b4, W5, b5, W6, b6):
    srcw = edge_index[0].reshape(NW, E // NW)
    dstw = edge_index[1].reshape(NW, E // NW)
    if PAD:
        wids = jnp.arange(NW, dtype=jnp.int32)[:, None]
        js = jnp.arange(PAD, dtype=jnp.int32)[None, :]
        spad = (wids * PAD + js) % N
        dpad = N + wids * PAD + js
        srcw = jnp.concatenate([srcw, spad], axis=1)
        dstw = jnp.concatenate([dstw, dpad], axis=1)
    src3 = srcw.reshape(NW, C, K)
    dst3 = dstw.reshape(NW, C, K)
    b1r, b2r, b3r = b1.reshape(1, -1), b2.reshape(1, -1), b3.reshape(1, -1)
    b4r, b5r, b6r = b4.reshape(1, -1), b5.reshape(1, -1), b6.reshape(1, -1)

    degP = _deg(dst3)
    dinv = _tc(_dinv_body, 1, degP)

    g1 = _tc(_t1_body, 32, x, W1, dinv)
    P = _agg32(g1, src3, dst3)
    g2 = _tc(_t2_body, 32, P, g1, b1r, dinv)
    P = _agg32(g2, src3, dst3)
    g3 = _tc(_t3_body, 64, P, g2, W2, b2r, dinv)
    P = _agg64(g3, src3, dst3)
    g4 = _tc(_t4_body, 64, P, g3, W3, b3r, W4, dinv)
    P = _agg64(g4, src3, dst3)
    g5 = _tc(_t5_body, 32, P, g4, b4r, W5, dinv)
    P = _agg32(g5, src3, dst3)
    g6 = _tc(_t2_body, 32, P, g5, b5r, dinv)
    P = _agg32(g6, src3, dst3)
    return _tc(_t7_body, 128, P, g6, W6, b6r, dinv)


# speedup vs baseline: 2.0622x; 1.0921x over previous
import functools

import jax
import jax.numpy as jnp
from jax import lax
from jax.experimental import pallas as pl
from jax.experimental.pallas import tpu as pltpu
from jax.experimental.pallas import tpu_sc as plsc

N = 10000
E = 320000
NC, NS, L = 2, 16, 16
NW = NC * NS
K = 80
NB = 8
C = 128
EPW = C * K
PAD = EPW - E // NW
NDUM = 512
NP = N + NDUM
RPT = 624
TAIL = N - NS * RPT
ZR = 208

_MESH = dict(core_axis_name="c", subcore_axis_name="s",
             num_cores=NC, num_subcores=NS)


def _fill_const(ref, rows, width, value):
    per_row = width // L

    def body(r, _):
        i = r // per_row
        k = (r % per_row) * L
        ref[i, pl.ds(k, L)] = jnp.full((L,), value, jnp.float32)
        return 0

    lax.fori_loop(0, rows * per_row, body, 0)


def _make_agg(F):
    def body(g_hbm, src_hbm, dst_hbm, out_hbm, *refs):
        c = lax.axis_index("c")
        s = lax.axis_index("s")
        wid = s * NC + c
        src_v, dst_v = refs[0], refs[1]
        bufs = refs[2:2 + NB]
        zb = refs[2 + NB]
        out_sh = refs[3 + NB]
        gsem = refs[4 + NB:4 + 2 * NB]
        ssem = refs[4 + 2 * NB:4 + 3 * NB]

        pltpu.sync_copy(src_hbm.at[wid], src_v)
        pltpu.sync_copy(dst_hbm.at[wid], dst_v)

        _fill_const(zb, ZR, F, 0.0)
        base = s * RPT
        for q in range(RPT // ZR):
            pltpu.sync_copy(zb, out_sh.at[pl.ds(base + q * ZR, ZR)])

        @pl.when(s == NS - 1)
        def _():
            pltpu.sync_copy(zb.at[pl.ds(0, TAIL)],
                            out_sh.at[pl.ds(NS * RPT, TAIL)])

        plsc.subcore_barrier()

        def start_gather(j, i):
            pltpu.async_copy(g_hbm.at[src_v.at[j]], bufs[i], gsem[i])

        def wait_gather(i):
            pltpu.make_async_copy(g_hbm.at[src_v.at[0]], bufs[i],
                                  gsem[i]).wait()

        def start_scatter(j, i):
            pltpu.async_copy(bufs[i], out_sh.at[dst_v.at[j]], ssem[i],
                             add=True)

        def wait_scatter(i):
            pltpu.make_async_copy(bufs[i], out_sh.at[dst_v.at[0]],
                                  ssem[i]).wait()

        for i in range(NB):
            start_gather(i, i)

        def block(t, _):
            for i in range(NB):
                wait_gather(i)
                start_scatter(t * NB + i, i)
            for i in range(NB):
                wait_scatter(i)
                start_gather((t + 1) * NB + i, i)
            return 0

        lax.fori_loop(0, C // NB - 1, block, 0)
        for i in range(NB):
            wait_gather(i)
            start_scatter(C - NB + i, i)
        for i in range(NB):
            wait_scatter(i)
        plsc.subcore_barrier()

        for q in range(RPT // ZR):
            o = base + q * ZR
            pltpu.sync_copy(out_sh.at[pl.ds(o, ZR)], out_hbm.at[c, pl.ds(o, ZR)])

        @pl.when(s == NS - 1)
        def _():
            pltpu.sync_copy(out_sh.at[pl.ds(NS * RPT, TAIL)],
                            out_hbm.at[c, pl.ds(NS * RPT, TAIL)])

    return pl.kernel(
        body,
        out_type=jax.ShapeDtypeStruct((NC, N, F), jnp.float32),
        mesh=plsc.VectorSubcoreMesh(**_MESH),
        compiler_params=pltpu.CompilerParams(use_tc_tiling_on_sc=False),
        scratch_types=(
            [pltpu.VMEM((C, K), jnp.int32),
             pltpu.VMEM((C, K), jnp.int32)]
            + [pltpu.VMEM((K, F), jnp.float32) for _ in range(NB)]
            + [pltpu.VMEM((ZR, F), jnp.float32),
               pltpu.VMEM_SHARED((NP, F), jnp.float32)]
            + [pltpu.SemaphoreType.DMA for _ in range(2 * NB)]
        ),
    )


def _make_deg():
    F = L

    def body(dst_hbm, out_hbm, dst_v, ones_v, zb, out_sh, sem):
        c = lax.axis_index("c")
        s = lax.axis_index("s")
        wid = s * NC + c

        pltpu.sync_copy(dst_hbm.at[wid], dst_v)
        _fill_const(ones_v, K, F, 1.0)
        _fill_const(zb, ZR, F, 0.0)
        base = s * RPT
        for q in range(RPT // ZR):
            pltpu.sync_copy(zb, out_sh.at[pl.ds(base + q * ZR, ZR)])

        @pl.when(s == NS - 1)
        def _():
            pltpu.sync_copy(zb.at[pl.ds(0, TAIL)],
                            out_sh.at[pl.ds(NS * RPT, TAIL)])

        plsc.subcore_barrier()

        def chunk(j, _):
            pltpu.sync_copy(ones_v, out_sh.at[dst_v.at[j]], add=True)
            return 0

        lax.fori_loop(0, C, chunk, 0)
        plsc.subcore_barrier()

        for q in range(RPT // ZR):
            o = base + q * ZR
            pltpu.sync_copy(out_sh.at[pl.ds(o, ZR)], out_hbm.at[c, pl.ds(o, ZR)])

        @pl.when(s == NS - 1)
        def _():
            pltpu.sync_copy(out_sh.at[pl.ds(NS * RPT, TAIL)],
                            out_hbm.at[c, pl.ds(NS * RPT, TAIL)])

    return pl.kernel(
        body,
        out_type=jax.ShapeDtypeStruct((NC, N, F), jnp.float32),
        mesh=plsc.VectorSubcoreMesh(**_MESH),
        compiler_params=pltpu.CompilerParams(use_tc_tiling_on_sc=False),
        scratch_types=[
            pltpu.VMEM((C, K), jnp.int32),
            pltpu.VMEM((K, F), jnp.float32),
            pltpu.VMEM((ZR, F), jnp.float32),
            pltpu.VMEM_SHARED((NP, F), jnp.float32),
            pltpu.SemaphoreType.DMA,
        ],
    )


_agg32 = _make_agg(32)
_agg64 = _make_agg(64)
_deg = _make_deg()


def _tc(body, out_dim, *args):
    return pl.pallas_call(
        body,
        out_shape=jax.ShapeDtypeStruct((N, out_dim), jnp.float32),
    )(*args)


def _dinv_body(deg_ref, o_ref):
    d = deg_ref[0, :, 0:1] + deg_ref[1, :, 0:1] + 1.0
    o_ref[...] = lax.rsqrt(d)


def _t1_body(x_ref, w_ref, dinv_ref, o_ref):
    o_ref[...] = jnp.dot(x_ref[...], w_ref[...],
                         preferred_element_type=jnp.float32) * dinv_ref[...]


def _t2_body(p_ref, g_ref, b_ref, dinv_ref, o_ref):
    dinv = dinv_ref[...]
    h = jnp.maximum(dinv * (p_ref[0] + p_ref[1] + g_ref[...]) + b_ref[...], 0.0)
    o_ref[...] = h * dinv


def _t3_body(p_ref, g_ref, w_ref, b_ref, dinv_ref, o_ref):
    dinv = dinv_ref[...]
    u = dinv * (p_ref[0] + p_ref[1] + g_ref[...])
    h = jnp.maximum(jnp.dot(u, w_ref[...],
                    preferred_element_type=jnp.float32) + b_ref[...], 0.0)
    o_ref[...] = h * dinv


def _t4_body(p_ref, g_ref, w3_ref, b_ref, w4_ref, dinv_ref, o_ref):
    dinv = dinv_ref[...]
    u = dinv * (p_ref[0] + p_ref[1] + g_ref[...])
    h = jnp.maximum(jnp.dot(u, w3_ref[...],
                    preferred_element_type=jnp.float32) + b_ref[...], 0.0)
    o_ref[...] = jnp.dot(h, w4_ref[...],
                         preferred_element_type=jnp.float32) * dinv


def _t5_body(p_ref, g_ref, b_ref, w_ref, dinv_ref, o_ref):
    dinv = dinv_ref[...]
    h = jnp.maximum(dinv * (p_ref[0] + p_ref[1] + g_ref[...]) + b_ref[...], 0.0)
    o_ref[...] = jnp.dot(h, w_ref[...],
                         preferred_element_type=jnp.float32) * dinv


def _t7_body(p_ref, g_ref, w_ref, b_ref, dinv_ref, o_ref):
    u = dinv_ref[...] * (p_ref[0] + p_ref[1] + g_ref[...])
    z = jnp.dot(u, w_ref[...], preferred_element_type=jnp.float32) + b_ref[...]
    o_ref[...] = jax.nn.sigmoid(z)


def kernel(x, edge_index, W1, b1, W2, b2, W3, b3, W4, b4, W5, b5, W6, b6):
    srcw = edge_index[0].reshape(NW, E // NW)
    dstw = edge_index[1].reshape(NW, E // NW)
    if PAD:
        wids = jnp.arange(NW, dtype=jnp.int32)[:, None]
        js = jnp.arange(PAD, dtype=jnp.int32)[None, :]
        spad = (wids * PAD + js) % N
        dpad = N + (wids * PAD + js) % NDUM
        srcw = jnp.concatenate([srcw, spad], axis=1)
        dstw = jnp.concatenate([dstw, dpad], axis=1)
    src3 = srcw.reshape(NW, C, K)
    dst3 = dstw.reshape(NW, C, K)
    b1r, b2r, b3r = b1.reshape(1, -1), b2.reshape(1, -1), b3.reshape(1, -1)
    b4r, b5r, b6r = b4.reshape(1, -1), b5.reshape(1, -1), b6.reshape(1, -1)

    degP = _deg(dst3)
    dinv = _tc(_dinv_body, 1, degP)

    g1 = _tc(_t1_body, 32, x, W1, dinv)
    P = _agg32(g1, src3, dst3)
    g2 = _tc(_t2_body, 32, P, g1, b1r, dinv)
    P = _agg32(g2, src3, dst3)
    g3 = _tc(_t3_body, 64, P, g2, W2, b2r, dinv)
    P = _agg64(g3, src3, dst3)
    g4 = _tc(_t4_body, 64, P, g3, W3, b3r, W4, dinv)
    P = _agg64(g4, src3, dst3)
    g5 = _tc(_t5_body, 32, P, g4, b4r, W5, dinv)
    P = _agg32(g5, src3, dst3)
    g6 = _tc(_t2_body, 32, P, g5, b5r, dinv)
    P = _agg32(g6, src3, dst3)
    return _tc(_t7_body, 128, P, g6, W6, b6r, dinv)


# speedup vs baseline: 2.0823x; 1.0097x over previous
import functools

import jax
import jax.numpy as jnp
from jax import lax
from jax.experimental import pallas as pl
from jax.experimental.pallas import tpu as pltpu
from jax.experimental.pallas import tpu_sc as plsc

N = 10000
E = 320000
NC, NS, L = 2, 16, 16
NW = NC * NS
EPW = 10240
PAD = EPW - E // NW
NDUM = 512
NP = N + NDUM
RPT = 624
TAIL = N - NS * RPT
ZR = 208

_MESH = dict(core_axis_name="c", subcore_axis_name="s",
             num_cores=NC, num_subcores=NS)


def _fill_const(ref, rows, width, value):
    per_row = width // L

    def body(r, _):
        i = r // per_row
        k = (r % per_row) * L
        ref[i, pl.ds(k, L)] = jnp.full((L,), value, jnp.float32)
        return 0

    lax.fori_loop(0, rows * per_row, body, 0)


def _make_agg(F, K, NB):
    C = EPW // K

    def body(g_hbm, src_hbm, dst_hbm, out_hbm, *refs):
        c = lax.axis_index("c")
        s = lax.axis_index("s")
        wid = s * NC + c
        src_v, dst_v = refs[0], refs[1]
        bufs = refs[2:2 + NB]
        zb = refs[2 + NB]
        out_sh = refs[3 + NB]
        gsem = refs[4 + NB:4 + 2 * NB]
        ssem = refs[4 + 2 * NB:4 + 3 * NB]

        pltpu.sync_copy(src_hbm.at[wid], src_v)
        pltpu.sync_copy(dst_hbm.at[wid], dst_v)

        _fill_const(zb, ZR, F, 0.0)
        base = s * RPT
        for q in range(RPT // ZR):
            pltpu.sync_copy(zb, out_sh.at[pl.ds(base + q * ZR, ZR)])

        @pl.when(s == NS - 1)
        def _():
            pltpu.sync_copy(zb.at[pl.ds(0, TAIL)],
                            out_sh.at[pl.ds(NS * RPT, TAIL)])

        plsc.subcore_barrier()

        def start_gather(j, i):
            pltpu.async_copy(g_hbm.at[src_v.at[j]], bufs[i], gsem[i])

        def wait_gather(i):
            pltpu.make_async_copy(g_hbm.at[src_v.at[0]], bufs[i],
                                  gsem[i]).wait()

        def start_scatter(j, i):
            pltpu.async_copy(bufs[i], out_sh.at[dst_v.at[j]], ssem[i],
                             add=True)

        def wait_scatter(i):
            pltpu.make_async_copy(bufs[i], out_sh.at[dst_v.at[0]],
                                  ssem[i]).wait()

        for i in range(NB):
            start_gather(i, i)

        def block(t, _):
            for i in range(NB):
                wait_gather(i)
                start_scatter(t * NB + i, i)
            for i in range(NB):
                wait_scatter(i)
                start_gather((t + 1) * NB + i, i)
            return 0

        lax.fori_loop(0, C // NB - 1, block, 0)
        for i in range(NB):
            wait_gather(i)
            start_scatter(C - NB + i, i)
        for i in range(NB):
            wait_scatter(i)
        plsc.subcore_barrier()

        for q in range(RPT // ZR):
            o = base + q * ZR
            pltpu.sync_copy(out_sh.at[pl.ds(o, ZR)], out_hbm.at[c, pl.ds(o, ZR)])

        @pl.when(s == NS - 1)
        def _():
            pltpu.sync_copy(out_sh.at[pl.ds(NS * RPT, TAIL)],
                            out_hbm.at[c, pl.ds(NS * RPT, TAIL)])

    return pl.kernel(
        body,
        out_type=jax.ShapeDtypeStruct((NC, N, F), jnp.float32),
        mesh=plsc.VectorSubcoreMesh(**_MESH),
        compiler_params=pltpu.CompilerParams(use_tc_tiling_on_sc=False),
        scratch_types=(
            [pltpu.VMEM((C, K), jnp.int32),
             pltpu.VMEM((C, K), jnp.int32)]
            + [pltpu.VMEM((K, F), jnp.float32) for _ in range(NB)]
            + [pltpu.VMEM((ZR, F), jnp.float32),
               pltpu.VMEM_SHARED((NP, F), jnp.float32)]
            + [pltpu.SemaphoreType.DMA for _ in range(2 * NB)]
        ),
    )


def _make_deg(K=80):
    F = L
    C = EPW // K

    def body(dst_hbm, out_hbm, dst_v, ones_v, zb, out_sh, sem):
        c = lax.axis_index("c")
        s = lax.axis_index("s")
        wid = s * NC + c

        pltpu.sync_copy(dst_hbm.at[wid], dst_v)
        _fill_const(ones_v, K, F, 1.0)
        _fill_const(zb, ZR, F, 0.0)
        base = s * RPT
        for q in range(RPT // ZR):
            pltpu.sync_copy(zb, out_sh.at[pl.ds(base + q * ZR, ZR)])

        @pl.when(s == NS - 1)
        def _():
            pltpu.sync_copy(zb.at[pl.ds(0, TAIL)],
                            out_sh.at[pl.ds(NS * RPT, TAIL)])

        plsc.subcore_barrier()

        def chunk(j, _):
            pltpu.sync_copy(ones_v, out_sh.at[dst_v.at[j]], add=True)
            return 0

        lax.fori_loop(0, C, chunk, 0)
        plsc.subcore_barrier()

        for q in range(RPT // ZR):
            o = base + q * ZR
            pltpu.sync_copy(out_sh.at[pl.ds(o, ZR)], out_hbm.at[c, pl.ds(o, ZR)])

        @pl.when(s == NS - 1)
        def _():
            pltpu.sync_copy(out_sh.at[pl.ds(NS * RPT, TAIL)],
                            out_hbm.at[c, pl.ds(NS * RPT, TAIL)])

    return pl.kernel(
        body,
        out_type=jax.ShapeDtypeStruct((NC, N, F), jnp.float32),
        mesh=plsc.VectorSubcoreMesh(**_MESH),
        compiler_params=pltpu.CompilerParams(use_tc_tiling_on_sc=False),
        scratch_types=[
            pltpu.VMEM((C, K), jnp.int32),
            pltpu.VMEM((K, F), jnp.float32),
            pltpu.VMEM((ZR, F), jnp.float32),
            pltpu.VMEM_SHARED((NP, F), jnp.float32),
            pltpu.SemaphoreType.DMA,
        ],
    )


_agg32 = _make_agg(32, K=128, NB=8)
_agg64 = _make_agg(64, K=80, NB=8)
_deg = _make_deg(K=80)


def _tc(body, out_dim, *args):
    return pl.pallas_call(
        body,
        out_shape=jax.ShapeDtypeStruct((N, out_dim), jnp.float32),
    )(*args)


def _dinv_body(deg_ref, o_ref):
    d = deg_ref[0, :, 0:1] + deg_ref[1, :, 0:1] + 1.0
    o_ref[...] = lax.rsqrt(d)


def _t1_body(x_ref, w_ref, dinv_ref, o_ref):
    o_ref[...] = jnp.dot(x_ref[...], w_ref[...],
                         preferred_element_type=jnp.float32) * dinv_ref[...]


def _t2_body(p_ref, g_ref, b_ref, dinv_ref, o_ref):
    dinv = dinv_ref[...]
    h = jnp.maximum(dinv * (p_ref[0] + p_ref[1] + g_ref[...]) + b_ref[...], 0.0)
    o_ref[...] = h * dinv


def _t3_body(p_ref, g_ref, w_ref, b_ref, dinv_ref, o_ref):
    dinv = dinv_ref[...]
    u = dinv * (p_ref[0] + p_ref[1] + g_ref[...])
    h = jnp.maximum(jnp.dot(u, w_ref[...],
                    preferred_element_type=jnp.float32) + b_ref[...], 0.0)
    o_ref[...] = h * dinv


def _t4_body(p_ref, g_ref, w3_ref, b_ref, w4_ref, dinv_ref, o_ref):
    dinv = dinv_ref[...]
    u = dinv * (p_ref[0] + p_ref[1] + g_ref[...])
    h = jnp.maximum(jnp.dot(u, w3_ref[...],
                    preferred_element_type=jnp.float32) + b_ref[...], 0.0)
    o_ref[...] = jnp.dot(h, w4_ref[...],
                         preferred_element_type=jnp.float32) * dinv


def _t5_body(p_ref, g_ref, b_ref, w_ref, dinv_ref, o_ref):
    dinv = dinv_ref[...]
    h = jnp.maximum(dinv * (p_ref[0] + p_ref[1] + g_ref[...]) + b_ref[...], 0.0)
    o_ref[...] = jnp.dot(h, w_ref[...],
                         preferred_element_type=jnp.float32) * dinv


def _t7_body(p_ref, g_ref, w_ref, b_ref, dinv_ref, o_ref):
    u = dinv_ref[...] * (p_ref[0] + p_ref[1] + g_ref[...])
    z = jnp.dot(u, w_ref[...], preferred_element_type=jnp.float32) + b_ref[...]
    o_ref[...] = jax.nn.sigmoid(z)


def kernel(x, edge_index, W1, b1, W2, b2, W3, b3, W4, b4, W5, b5, W6, b6):
    srcw = edge_index[0].reshape(NW, E // NW)
    dstw = edge_index[1].reshape(NW, E // NW)
    if PAD:
        wids = jnp.arange(NW, dtype=jnp.int32)[:, None]
        js = jnp.arange(PAD, dtype=jnp.int32)[None, :]
        spad = (wids * PAD + js) % N
        dpad = N + (wids * PAD + js) % NDUM
        srcw = jnp.concatenate([srcw, spad], axis=1)
        dstw = jnp.concatenate([dstw, dpad], axis=1)
    src_a = srcw.reshape(NW, EPW // 128, 128)
    dst_a = dstw.reshape(NW, EPW // 128, 128)
    src_b = srcw.reshape(NW, EPW // 80, 80)
    dst_b = dstw.reshape(NW, EPW // 80, 80)
    b1r, b2r, b3r = b1.reshape(1, -1), b2.reshape(1, -1), b3.reshape(1, -1)
    b4r, b5r, b6r = b4.reshape(1, -1), b5.reshape(1, -1), b6.reshape(1, -1)

    degP = _deg(dst_b)
    dinv = _tc(_dinv_body, 1, degP)

    g1 = _tc(_t1_body, 32, x, W1, dinv)
    P = _agg32(g1, src_a, dst_a)
    g2 = _tc(_t2_body, 32, P, g1, b1r, dinv)
    P = _agg32(g2, src_a, dst_a)
    g3 = _tc(_t3_body, 64, P, g2, W2, b2r, dinv)
    P = _agg64(g3, src_b, dst_b)
    g4 = _tc(_t4_body, 64, P, g3, W3, b3r, W4, dinv)
    P = _agg64(g4, src_b, dst_b)
    g5 = _tc(_t5_body, 32, P, g4, b4r, W5, dinv)
    P = _agg32(g5, src_a, dst_a)
    g6 = _tc(_t2_body, 32, P, g5, b5r, dinv)
    P = _agg32(g6, src_a, dst_a)
    return _tc(_t7_body, 128, P, g6, W6, b6r, dinv)


# speedup vs baseline: 2.1373x; 1.0264x over previous
import functools

import jax
import jax.numpy as jnp
from jax import lax
from jax.experimental import pallas as pl
from jax.experimental.pallas import tpu as pltpu
from jax.experimental.pallas import tpu_sc as plsc

N = 10000
E = 320000
NC, NS, L = 2, 16, 16
NW = NC * NS
EPW = 10240
PAD = EPW - E // NW
NDUM = 512
NP = N + NDUM
RPT = 624
TAIL = N - NS * RPT
ZR = 208

_MESH = dict(core_axis_name="c", subcore_axis_name="s",
             num_cores=NC, num_subcores=NS)


def _fill_const(ref, rows, width, value):
    per_row = width // L

    def body(r, _):
        i = r // per_row
        k = (r % per_row) * L
        ref[i, pl.ds(k, L)] = jnp.full((L,), value, jnp.float32)
        return 0

    lax.fori_loop(0, rows * per_row, body, 0)


def _make_agg(F, K, NB):
    C = EPW // K

    def body(g_hbm, src_hbm, dst_hbm, out_hbm, *refs):
        c = lax.axis_index("c")
        s = lax.axis_index("s")
        wid = s * NC + c
        src_v, dst_v = refs[0], refs[1]
        bufs = refs[2:2 + NB]
        zb = refs[2 + NB]
        out_sh = refs[3 + NB]
        gsem = refs[4 + NB:4 + 2 * NB]
        ssem = refs[4 + 2 * NB:4 + 3 * NB]

        pltpu.sync_copy(src_hbm.at[wid], src_v)
        pltpu.sync_copy(dst_hbm.at[wid], dst_v)

        _fill_const(zb, ZR, F, 0.0)
        base = s * RPT
        for q in range(RPT // ZR):
            pltpu.sync_copy(zb, out_sh.at[pl.ds(base + q * ZR, ZR)])

        @pl.when(s == NS - 1)
        def _():
            pltpu.sync_copy(zb.at[pl.ds(0, TAIL)],
                            out_sh.at[pl.ds(NS * RPT, TAIL)])

        plsc.subcore_barrier()

        def start_gather(j, i):
            pltpu.async_copy(g_hbm.at[src_v.at[j]], bufs[i], gsem[i])

        def wait_gather(i):
            pltpu.make_async_copy(g_hbm.at[src_v.at[0]], bufs[i],
                                  gsem[i]).wait()

        def start_scatter(j, i):
            pltpu.async_copy(bufs[i], out_sh.at[dst_v.at[j]], ssem[i],
                             add=True)

        def wait_scatter(i):
            pltpu.make_async_copy(bufs[i], out_sh.at[dst_v.at[0]],
                                  ssem[i]).wait()

        for i in range(NB):
            start_gather(i, i)

        def block(t, _):
            for i in range(NB):
                wait_gather(i)
                start_scatter(t * NB + i, i)
            for i in range(NB):
                wait_scatter(i)
                start_gather((t + 1) * NB + i, i)
            return 0

        lax.fori_loop(0, C // NB - 1, block, 0)
        for i in range(NB):
            wait_gather(i)
            start_scatter(C - NB + i, i)
        for i in range(NB):
            wait_scatter(i)
        plsc.subcore_barrier()

        for q in range(RPT // ZR):
            o = base + q * ZR
            pltpu.sync_copy(out_sh.at[pl.ds(o, ZR)], out_hbm.at[c, pl.ds(o, ZR)])

        @pl.when(s == NS - 1)
        def _():
            pltpu.sync_copy(out_sh.at[pl.ds(NS * RPT, TAIL)],
                            out_hbm.at[c, pl.ds(NS * RPT, TAIL)])

    return pl.kernel(
        body,
        out_type=jax.ShapeDtypeStruct((NC, N, F), jnp.float32),
        mesh=plsc.VectorSubcoreMesh(**_MESH),
        compiler_params=pltpu.CompilerParams(use_tc_tiling_on_sc=False),
        scratch_types=(
            [pltpu.VMEM((C, K), jnp.int32),
             pltpu.VMEM((C, K), jnp.int32)]
            + [pltpu.VMEM((K, F), jnp.float32) for _ in range(NB)]
            + [pltpu.VMEM((ZR, F), jnp.float32),
               pltpu.VMEM_SHARED((NP, F), jnp.float32)]
            + [pltpu.SemaphoreType.DMA for _ in range(2 * NB)]
        ),
    )


def _make_deg(K=80):
    F = L
    C = EPW // K

    def body(dst_hbm, out_hbm, dst_v, ones_v, zb, out_sh, sem):
        c = lax.axis_index("c")
        s = lax.axis_index("s")
        wid = s * NC + c

        pltpu.sync_copy(dst_hbm.at[wid], dst_v)
        _fill_const(ones_v, K, F, 1.0)
        _fill_const(zb, ZR, F, 0.0)
        base = s * RPT
        for q in range(RPT // ZR):
            pltpu.sync_copy(zb, out_sh.at[pl.ds(base + q * ZR, ZR)])

        @pl.when(s == NS - 1)
        def _():
            pltpu.sync_copy(zb.at[pl.ds(0, TAIL)],
                            out_sh.at[pl.ds(NS * RPT, TAIL)])

        plsc.subcore_barrier()

        DEPTH = 8

        def start(j):
            pltpu.async_copy(ones_v, out_sh.at[dst_v.at[j]], sem, add=True)

        def wait_one():
            pltpu.make_async_copy(ones_v, out_sh.at[dst_v.at[0]], sem).wait()

        for j in range(DEPTH):
            start(j)

        def blk(t, _):
            wait_one()
            start(t + DEPTH)
            return 0

        lax.fori_loop(0, C - DEPTH, blk, 0)
        for j in range(DEPTH):
            wait_one()
        plsc.subcore_barrier()

        for q in range(RPT // ZR):
            o = base + q * ZR
            pltpu.sync_copy(out_sh.at[pl.ds(o, ZR)], out_hbm.at[c, pl.ds(o, ZR)])

        @pl.when(s == NS - 1)
        def _():
            pltpu.sync_copy(out_sh.at[pl.ds(NS * RPT, TAIL)],
                            out_hbm.at[c, pl.ds(NS * RPT, TAIL)])

    return pl.kernel(
        body,
        out_type=jax.ShapeDtypeStruct((NC, N, F), jnp.float32),
        mesh=plsc.VectorSubcoreMesh(**_MESH),
        compiler_params=pltpu.CompilerParams(use_tc_tiling_on_sc=False),
        scratch_types=[
            pltpu.VMEM((C, K), jnp.int32),
            pltpu.VMEM((K, F), jnp.float32),
            pltpu.VMEM((ZR, F), jnp.float32),
            pltpu.VMEM_SHARED((NP, F), jnp.float32),
            pltpu.SemaphoreType.DMA,
        ],
    )


_agg32 = _make_agg(32, K=128, NB=8)
_agg64 = _make_agg(64, K=80, NB=8)
_deg = _make_deg(K=80)


def _tc(body, out_dim, *args):
    return pl.pallas_call(
        body,
        out_shape=jax.ShapeDtypeStruct((N, out_dim), jnp.float32),
    )(*args)


def _z1_body(x_ref, w_ref, o_ref):
    o_ref[...] = jnp.dot(x_ref[...], w_ref[...],
                         preferred_element_type=jnp.float32)


def _dinv_g1_body(deg_ref, z_ref, dinv_ref, g_ref):
    d = deg_ref[0, :, 0:1] + deg_ref[1, :, 0:1] + 1.0
    dinv = lax.rsqrt(d)
    dinv_ref[...] = dinv
    g_ref[...] = z_ref[...] * dinv


def _t2_body(p_ref, g_ref, b_ref, dinv_ref, o_ref):
    dinv = dinv_ref[...]
    h = jnp.maximum(dinv * (p_ref[0] + p_ref[1] + g_ref[...]) + b_ref[...], 0.0)
    o_ref[...] = h * dinv


def _t3_body(p_ref, g_ref, w_ref, b_ref, dinv_ref, o_ref):
    dinv = dinv_ref[...]
    u = dinv * (p_ref[0] + p_ref[1] + g_ref[...])
    h = jnp.maximum(jnp.dot(u, w_ref[...],
                    preferred_element_type=jnp.float32) + b_ref[...], 0.0)
    o_ref[...] = h * dinv


def _t4_body(p_ref, g_ref, w3_ref, b_ref, w4_ref, dinv_ref, o_ref):
    dinv = dinv_ref[...]
    u = dinv * (p_ref[0] + p_ref[1] + g_ref[...])
    h = jnp.maximum(jnp.dot(u, w3_ref[...],
                    preferred_element_type=jnp.float32) + b_ref[...], 0.0)
    o_ref[...] = jnp.dot(h, w4_ref[...],
                         preferred_element_type=jnp.float32) * dinv


def _t5_body(p_ref, g_ref, b_ref, w_ref, dinv_ref, o_ref):
    dinv = dinv_ref[...]
    h = jnp.maximum(dinv * (p_ref[0] + p_ref[1] + g_ref[...]) + b_ref[...], 0.0)
    o_ref[...] = jnp.dot(h, w_ref[...],
                         preferred_element_type=jnp.float32) * dinv


def _t7_body(p_ref, g_ref, w_ref, b_ref, dinv_ref, o_ref):
    u = dinv_ref[...] * (p_ref[0] + p_ref[1] + g_ref[...])
    z = jnp.dot(u, w_ref[...], preferred_element_type=jnp.float32) + b_ref[...]
    o_ref[...] = jax.nn.sigmoid(z)


def kernel(x, edge_index, W1, b1, W2, b2, W3, b3, W4, b4, W5, b5, W6, b6):
    srcw = edge_index[0].reshape(NW, E // NW)
    dstw = edge_index[1].reshape(NW, E // NW)
    if PAD:
        wids = jnp.arange(NW, dtype=jnp.int32)[:, None]
        js = jnp.arange(PAD, dtype=jnp.int32)[None, :]
        spad = (wids * PAD + js) % N
        dpad = N + (wids * PAD + js) % NDUM
        srcw = jnp.concatenate([srcw, spad], axis=1)
        dstw = jnp.concatenate([dstw, dpad], axis=1)
    src_a = srcw.reshape(NW, EPW // 128, 128)
    dst_a = dstw.reshape(NW, EPW // 128, 128)
    src_b = srcw.reshape(NW, EPW // 80, 80)
    dst_b = dstw.reshape(NW, EPW // 80, 80)
    b1r, b2r, b3r = b1.reshape(1, -1), b2.reshape(1, -1), b3.reshape(1, -1)
    b4r, b5r, b6r = b4.reshape(1, -1), b5.reshape(1, -1), b6.reshape(1, -1)

    z1 = _tc(_z1_body, 32, x, W1)
    degP = _deg(dst_b)
    dinv, g1 = pl.pallas_call(
        _dinv_g1_body,
        out_shape=(jax.ShapeDtypeStruct((N, 1), jnp.float32),
                   jax.ShapeDtypeStruct((N, 32), jnp.float32)),
    )(degP, z1)
    P = _agg32(g1, src_a, dst_a)
    g2 = _tc(_t2_body, 32, P, g1, b1r, dinv)
    P = _agg32(g2, src_a, dst_a)
    g3 = _tc(_t3_body, 64, P, g2, W2, b2r, dinv)
    P = _agg64(g3, src_b, dst_b)
    g4 = _tc(_t4_body, 64, P, g3, W3, b3r, W4, dinv)
    P = _agg64(g4, src_b, dst_b)
    g5 = _tc(_t5_body, 32, P, g4, b4r, W5, dinv)
    P = _agg32(g5, src_a, dst_a)
    g6 = _tc(_t2_body, 32, P, g5, b5r, dinv)
    P = _agg32(g6, src_a, dst_a)
    return _tc(_t7_body, 128, P, g6, W6, b6r, dinv)
